# Initial kernel scaffold; baseline (speedup 1.0000x reference)
import jax
import jax.numpy as jnp
from jax.experimental import pallas as pl


def kernel(cat_a, cat_b, cat_c, W_cat_a, W_cat_b, W_cat_c):
    raise NotImplementedError("write your pallas kernel here")


# speedup vs baseline: 2.2656x; 2.2656x over previous
import functools

import jax
import jax.numpy as jnp
from jax import lax
from jax.experimental import pallas as pl
from jax.experimental.pallas import tpu as pltpu
from jax.experimental.pallas import tpu_sc as plsc

N, C, D = 16384, 20, 64
B = N * C
NW = 32
BPW = B // NW
CS = 512
NCHUNK = BPW // CS

_mesh = plsc.VectorSubcoreMesh(core_axis_name="c", subcore_axis_name="s")


@functools.partial(
    pl.kernel,
    mesh=_mesh,
    out_type=[jax.ShapeDtypeStruct((B, D), jnp.float32)] * 3,
    scratch_types=[
        pltpu.VMEM((CS,), jnp.int32),
        pltpu.VMEM((CS, D), jnp.float32),
        pltpu.SemaphoreType.DMA,
    ],
    compiler_params=pltpu.CompilerParams(use_tc_tiling_on_sc=False),
)
def _emb3(idx_a, idx_b, idx_c, w_a, w_b, w_c,
          out_a, out_b, out_c, idx_v, rows_v, sem):
    wid = lax.axis_index("s") * 2 + lax.axis_index("c")
    base = wid * BPW
    for table, idx_hbm, out_hbm in ((w_a, idx_a, out_a),
                                    (w_b, idx_b, out_b),
                                    (w_c, idx_c, out_c)):
        def body(g, carry, table=table, idx_hbm=idx_hbm, out_hbm=out_hbm):
            off = base + g * CS
            pltpu.sync_copy(idx_hbm.at[pl.ds(off, CS)], idx_v)
            pltpu.async_copy(table.at[idx_v], rows_v, sem).wait()
            pltpu.sync_copy(rows_v, out_hbm.at[pl.ds(off, CS)])
            return carry
        lax.fori_loop(0, NCHUNK, body, 0)


def kernel(cat_a, cat_b, cat_c, W_cat_a, W_cat_b, W_cat_c):
    ia = cat_a.reshape(B).astype(jnp.int32)
    ib = cat_b.reshape(B).astype(jnp.int32)
    ic = cat_c.reshape(B).astype(jnp.int32)
    oa, ob, oc = _emb3(ia, ib, ic, W_cat_a, W_cat_b, W_cat_c)
    return (oa.reshape(N, C, D), ob.reshape(N, C, D), oc.reshape(N, C, D))


# speedup vs baseline: 2.3315x; 1.0291x over previous
import functools

import jax
import jax.numpy as jnp
from jax import lax
from jax.experimental import pallas as pl
from jax.experimental.pallas import tpu as pltpu
from jax.experimental.pallas import tpu_sc as plsc

N, C, D = 16384, 20, 64
B = N * C
NW = 32
BPW = B // NW
CS = 512
NCHUNK = BPW // CS

_mesh = plsc.VectorSubcoreMesh(core_axis_name="c", subcore_axis_name="s")


@functools.partial(
    pl.kernel,
    mesh=_mesh,
    out_type=[jax.ShapeDtypeStruct((B, D), jnp.float32)] * 3,
    scratch_types=[
        pltpu.VMEM((BPW,), jnp.int32),
        pltpu.VMEM((CS, D), jnp.float32),
        pltpu.VMEM((CS, D), jnp.float32),
        pltpu.SemaphoreType.DMA,
        pltpu.SemaphoreType.DMA,
        pltpu.SemaphoreType.DMA,
        pltpu.SemaphoreType.DMA,
    ],
    compiler_params=pltpu.CompilerParams(use_tc_tiling_on_sc=False),
)
def _emb3(idx_a, idx_b, idx_c, w_a, w_b, w_c,
          out_a, out_b, out_c,
          idx_all, rows0, rows1, gsem0, gsem1, wsem0, wsem1):
    wid = lax.axis_index("s") * 2 + lax.axis_index("c")
    base = wid * BPW
    rows = (rows0, rows1)
    gsem = (gsem0, gsem1)
    wsem = (wsem0, wsem1)

    for table, idx_hbm, out_hbm in ((w_a, idx_a, out_a),
                                    (w_b, idx_b, out_b),
                                    (w_c, idx_c, out_c)):
        pltpu.sync_copy(idx_hbm.at[pl.ds(base, BPW)], idx_all)

        def gather_desc(g, s, table=table):
            return pltpu.make_async_copy(
                table.at[idx_all.at[pl.ds(g * CS, CS)]], rows[s], gsem[s])

        def write_desc(g, s, out_hbm=out_hbm):
            return pltpu.make_async_copy(
                rows[s], out_hbm.at[pl.ds(base + g * CS, CS)], wsem[s])

        gather_desc(0, 0).start()
        gather_desc(1, 1).start()
        gather_desc(0, 0).wait()
        write_desc(0, 0).start()

        def body(t, carry):
            g0 = 2 + 2 * t
            write_desc(g0 - 2, 0).wait()
            gather_desc(g0, 0).start()
            gather_desc(g0 - 1, 1).wait()
            write_desc(g0 - 1, 1).start()
            write_desc(g0 - 1, 1).wait()
            gather_desc(g0 + 1, 1).start()
            gather_desc(g0, 0).wait()
            write_desc(g0, 0).start()
            return carry

        lax.fori_loop(0, (NCHUNK - 2) // 2, body, 0)

        gather_desc(NCHUNK - 1, 1).wait()
        write_desc(NCHUNK - 1, 1).start()
        write_desc(NCHUNK - 2, 0).wait()
        write_desc(NCHUNK - 1, 1).wait()


def kernel(cat_a, cat_b, cat_c, W_cat_a, W_cat_b, W_cat_c):
    ia = cat_a.reshape(B).astype(jnp.int32)
    ib = cat_b.reshape(B).astype(jnp.int32)
    ic = cat_c.reshape(B).astype(jnp.int32)
    oa, ob, oc = _emb3(ia, ib, ic, W_cat_a, W_cat_b, W_cat_c)
    return (oa.reshape(N, C, D), ob.reshape(N, C, D), oc.reshape(N, C, D))


# speedup vs baseline: 2.4542x; 1.0526x over previous
import functools

import jax
import jax.numpy as jnp
from jax import lax
from jax.experimental import pallas as pl
from jax.experimental.pallas import tpu as pltpu
from jax.experimental.pallas import tpu_sc as plsc

N, C, D = 16384, 20, 64
B = N * C
NW = 32
BPW = B // NW
CS = 160
NCHUNK = BPW // CS
VC = 1000

_mesh = plsc.VectorSubcoreMesh(core_axis_name="c", subcore_axis_name="s")


@functools.partial(
    pl.kernel,
    mesh=_mesh,
    out_type=[jax.ShapeDtypeStruct((B, D), jnp.float32)] * 3,
    scratch_types=[
        pltpu.VMEM_SHARED((VC, D), jnp.float32),
        [pltpu.VMEM((BPW,), jnp.int32) for _ in range(3)],
        [[pltpu.VMEM((CS, D), jnp.float32) for _ in range(3)]
         for _ in range(2)],
        [[pltpu.SemaphoreType.DMA for _ in range(3)] for _ in range(2)],
        [[pltpu.SemaphoreType.DMA for _ in range(3)] for _ in range(2)],
    ],
    compiler_params=pltpu.CompilerParams(use_tc_tiling_on_sc=False),
)
def _emb3(idx_a, idx_b, idx_c, w_a, w_b, w_c,
          out_a, out_b, out_c,
          w_c_sh, idxs, rows, gsem, wsem):
    wid = lax.axis_index("s") * 2 + lax.axis_index("c")
    base = wid * BPW

    @pl.when(lax.axis_index("s") == 0)
    def _stage():
        pltpu.sync_copy(w_c, w_c_sh)

    for t, idx_hbm in enumerate((idx_a, idx_b, idx_c)):
        pltpu.sync_copy(idx_hbm.at[pl.ds(base, BPW)], idxs[t])
    plsc.subcore_barrier()

    tables = (w_a, w_b, w_c_sh)
    outs = (out_a, out_b, out_c)

    def gather_desc(g, p, t):
        return pltpu.make_async_copy(
            tables[t].at[idxs[t].at[pl.ds(g * CS, CS)]],
            rows[p][t], gsem[p][t])

    def write_desc(g, p, t):
        return pltpu.make_async_copy(
            rows[p][t], outs[t].at[pl.ds(base + g * CS, CS)], wsem[p][t])

    def start_g(g, p):
        for t in range(3):
            gather_desc(g, p, t).start()

    def wait_g(g, p):
        for t in range(3):
            gather_desc(g, p, t).wait()

    def start_w(g, p):
        for t in range(3):
            write_desc(g, p, t).start()

    def wait_w(g, p):
        for t in range(3):
            write_desc(g, p, t).wait()

    start_g(0, 0)
    start_g(1, 1)
    wait_g(0, 0)
    start_w(0, 0)

    def body(tt, carry):
        g0 = 2 * tt
        wait_w(g0 - 2, 0)
        start_g(g0, 0)
        wait_g(g0 - 1, 1)
        start_w(g0 - 1, 1)
        wait_w(g0 - 1, 1)
        start_g(g0 + 1, 1)
        wait_g(g0, 0)
        start_w(g0, 0)
        return carry

    lax.fori_loop(1, NCHUNK // 2, body, 0)

    wait_g(NCHUNK - 1, 1)
    start_w(NCHUNK - 1, 1)
    wait_w(NCHUNK - 2, 0)
    wait_w(NCHUNK - 1, 1)


def kernel(cat_a, cat_b, cat_c, W_cat_a, W_cat_b, W_cat_c):
    ia = cat_a.reshape(B).astype(jnp.int32)
    ib = cat_b.reshape(B).astype(jnp.int32)
    ic = cat_c.reshape(B).astype(jnp.int32)
    oa, ob, oc = _emb3(ia, ib, ic, W_cat_a, W_cat_b, W_cat_c)
    return (oa.reshape(N, C, D), ob.reshape(N, C, D), oc.reshape(N, C, D))
